# baseline (speedup 1.0000x reference)
import functools

import jax
import jax.numpy as jnp
import numpy as np
from jax.experimental import pallas as pl
from jax.experimental.pallas import tpu as pltpu

B, S, DM = 1, 2048, 2048
H, D, RD, TOPK = 12, 64, 32, 512
BQ = 256


def _swap_pairs(x):
    ncols = x.shape[-1]
    col = jax.lax.broadcasted_iota(jnp.int32, x.shape, x.ndim - 1)
    fwd = pltpu.roll(x, ncols - 1, axis=x.ndim - 1)
    bwd = pltpu.roll(x, 1, axis=x.ndim - 1)
    return jnp.where(col % 2 == 0, fwd, bwd)


def _stage_a(hid_ref, wq_ref, wk_ref, ww_ref, cq_ref, sq_ref, ck_ref, sk_ref,
             q_ref, k_ref, w_ref):
    h = hid_ref[...]
    q = jnp.dot(h, wq_ref[...], preferred_element_type=jnp.float32)
    q_ref[...] = q * cq_ref[...] + _swap_pairs(q) * sq_ref[...]
    k = jnp.dot(h, wk_ref[...], preferred_element_type=jnp.float32)
    k_ref[...] = k * ck_ref[...] + _swap_pairs(k) * sk_ref[...]
    wl = jnp.dot(h, ww_ref[...], preferred_element_type=jnp.float32)
    wl = wl - jnp.max(wl, axis=-1, keepdims=True)
    e = jnp.exp(wl)
    w_ref[...] = e / jnp.sum(e, axis=-1, keepdims=True) * (D ** -0.5)


def _stage_b(q_ref, kt_ref, w_ref, s_ref):
    q = q_ref[...]
    w = w_ref[...]
    kt = kt_ref[...]
    acc = jnp.zeros((BQ, S), jnp.float32)
    for h in range(H):
        lg = jnp.dot(q[:, h * D:(h + 1) * D], kt,
                     preferred_element_type=jnp.float32)
        acc = acc + jnp.maximum(lg, 0.0) * w[:, h:h + 1]
    row = pl.program_id(0) * BQ + jax.lax.broadcasted_iota(jnp.int32, (BQ, S), 0)
    col = jax.lax.broadcasted_iota(jnp.int32, (BQ, S), 1)
    colf = col.astype(jnp.float32)
    valid = jnp.where(acc == 0.0, colf * jnp.float32(-1e-6), acc)
    s_ref[...] = jnp.where(col <= row, valid, -colf)


BS = 256


def _bitonic_step(v, ix, d, k):
    n = v.shape[-1]
    col = jax.lax.broadcasted_iota(jnp.int32, v.shape, v.ndim - 1)
    lower = (col & d) == 0
    fv = pltpu.roll(v, n - d, axis=v.ndim - 1)
    bv = pltpu.roll(v, d, axis=v.ndim - 1)
    pv = jnp.where(lower, fv, bv)
    fi = pltpu.roll(ix, n - d, axis=ix.ndim - 1)
    bi = pltpu.roll(ix, d, axis=ix.ndim - 1)
    pix = jnp.where(lower, fi, bi)
    want_max = (((col // d) ^ (col // k)) & 1) == 0
    hi = jnp.where(want_max, pv, v)
    lo = jnp.where(want_max, v, pv)
    take_p = hi > lo
    return jnp.where(take_p, pv, v), jnp.where(take_p, pix, ix)


def _bitonic_sort_desc(v, ix):
    n = v.shape[-1]
    k = 2
    while k <= n:
        d = k // 2
        while d >= 1:
            v, ix = _bitonic_step(v, ix, d, k)
            d //= 2
        k *= 2
    return v, ix


def _stage_c(s_ref, tv_ref, ti_ref):
    v = s_ref[...]
    ix = jax.lax.broadcasted_iota(jnp.int32, v.shape, 1)
    v, ix = _bitonic_sort_desc(v, ix)
    tv = v[:, :TOPK]
    tv_ref[...] = jnp.where(tv < -0.5, jnp.float32(-1e9),
                            jnp.where(tv < 0.0, jnp.float32(0.0), tv))
    ti_ref[...] = ix[:, :TOPK]


def _rope_tables(cos, sin):
    half = RD // 2
    c = cos[:, :half]
    s = sin[:, :half]
    cosF = jnp.repeat(c, 2, axis=1)
    sinF = jnp.stack([-s, s], axis=-1).reshape(S, RD)
    ones = jnp.ones((S, D - RD), jnp.float32)
    zeros = jnp.zeros((S, D - RD), jnp.float32)
    cos64 = jnp.concatenate([ones, cosF], axis=1)
    sin64 = jnp.concatenate([zeros, sinF], axis=1)
    cosQ = jnp.tile(cos64, (1, H))
    sinQ = jnp.tile(sin64, (1, H))
    return cosQ, sinQ, cos64, sin64


@jax.jit
def kernel(hidden_states, cos, sin, wq, wk, ww):
    hid = hidden_states[0]
    cosQ, sinQ, cosK, sinK = _rope_tables(cos[0], sin[0])

    nblk = S // BQ
    q_rope, k_rope, w = pl.pallas_call(
        _stage_a,
        grid=(nblk,),
        in_specs=[
            pl.BlockSpec((BQ, DM), lambda i: (i, 0)),
            pl.BlockSpec((DM, H * D), lambda i: (0, 0)),
            pl.BlockSpec((DM, D), lambda i: (0, 0)),
            pl.BlockSpec((DM, H), lambda i: (0, 0)),
            pl.BlockSpec((BQ, H * D), lambda i: (i, 0)),
            pl.BlockSpec((BQ, H * D), lambda i: (i, 0)),
            pl.BlockSpec((BQ, D), lambda i: (i, 0)),
            pl.BlockSpec((BQ, D), lambda i: (i, 0)),
        ],
        out_specs=[
            pl.BlockSpec((BQ, H * D), lambda i: (i, 0)),
            pl.BlockSpec((BQ, D), lambda i: (i, 0)),
            pl.BlockSpec((BQ, H), lambda i: (i, 0)),
        ],
        out_shape=[
            jax.ShapeDtypeStruct((S, H * D), jnp.float32),
            jax.ShapeDtypeStruct((S, D), jnp.float32),
            jax.ShapeDtypeStruct((S, H), jnp.float32),
        ],
    )(hid, wq, wk, ww, cosQ, sinQ, cosK, sinK)

    kt = k_rope.T

    scores = pl.pallas_call(
        _stage_b,
        grid=(nblk,),
        in_specs=[
            pl.BlockSpec((BQ, H * D), lambda i: (i, 0)),
            pl.BlockSpec((D, S), lambda i: (0, 0)),
            pl.BlockSpec((BQ, H), lambda i: (i, 0)),
        ],
        out_specs=pl.BlockSpec((BQ, S), lambda i: (i, 0)),
        out_shape=jax.ShapeDtypeStruct((S, S), jnp.float32),
    )(q_rope, kt, w)

    tv, ti = pl.pallas_call(
        _stage_c,
        grid=(S // BS,),
        in_specs=[pl.BlockSpec((BS, S), lambda i: (i, 0))],
        out_specs=[
            pl.BlockSpec((BS, TOPK), lambda i: (i, 0)),
            pl.BlockSpec((BS, TOPK), lambda i: (i, 0)),
        ],
        out_shape=[
            jax.ShapeDtypeStruct((S, TOPK), jnp.float32),
            jax.ShapeDtypeStruct((S, TOPK), jnp.int32),
        ],
    )(scores)
    return tv[None], ti[None]


# speedup vs baseline: 1.1293x; 1.1293x over previous
import functools

import jax
import jax.numpy as jnp
import numpy as np
from jax.experimental import pallas as pl
from jax.experimental.pallas import tpu as pltpu

B, S, DM = 1, 2048, 2048
H, D, RD, TOPK = 12, 64, 32, 512
BQ = 256


def _swap_pairs(x):
    ncols = x.shape[-1]
    col = jax.lax.broadcasted_iota(jnp.int32, x.shape, x.ndim - 1)
    fwd = pltpu.roll(x, ncols - 1, axis=x.ndim - 1)
    bwd = pltpu.roll(x, 1, axis=x.ndim - 1)
    return jnp.where(col % 2 == 0, fwd, bwd)


def _stage_a(hid_ref, wq_ref, wk_ref, ww_ref, cq_ref, sq_ref, ck_ref, sk_ref,
             q_ref, k_ref, w_ref):
    h = hid_ref[...]
    q = jnp.dot(h, wq_ref[...], preferred_element_type=jnp.float32)
    q_ref[...] = q * cq_ref[...] + _swap_pairs(q) * sq_ref[...]
    k = jnp.dot(h, wk_ref[...], preferred_element_type=jnp.float32)
    k_ref[...] = k * ck_ref[...] + _swap_pairs(k) * sk_ref[...]
    wl = jnp.dot(h, ww_ref[...], preferred_element_type=jnp.float32)
    wl = wl - jnp.max(wl, axis=-1, keepdims=True)
    e = jnp.exp(wl)
    w_ref[...] = e / jnp.sum(e, axis=-1, keepdims=True) * (D ** -0.5)


def _stage_b(q_ref, kt_ref, w_ref, s_ref):
    q = q_ref[...]
    w = w_ref[...]
    kt = kt_ref[...]
    acc = jnp.zeros((BQ, S), jnp.float32)
    for h in range(H):
        lg = jnp.dot(q[:, h * D:(h + 1) * D], kt,
                     preferred_element_type=jnp.float32)
        acc = acc + jnp.maximum(lg, 0.0) * w[:, h:h + 1]
    row = pl.program_id(0) * BQ + jax.lax.broadcasted_iota(jnp.int32, (BQ, S), 0)
    col = jax.lax.broadcasted_iota(jnp.int32, (BQ, S), 1)
    colf = col.astype(jnp.float32)
    valid = jnp.where(acc == 0.0, colf * jnp.float32(-1e-6), acc)
    s_ref[...] = jnp.where(col <= row, valid, -colf)


BS = 256


def _bitonic_step(v, ix, d, k):
    n = v.shape[-1]
    col = jax.lax.broadcasted_iota(jnp.int32, v.shape, v.ndim - 1)
    lower = (col & d) == 0
    fv = pltpu.roll(v, n - d, axis=v.ndim - 1)
    bv = pltpu.roll(v, d, axis=v.ndim - 1)
    pv = jnp.where(lower, fv, bv)
    fi = pltpu.roll(ix, n - d, axis=ix.ndim - 1)
    bi = pltpu.roll(ix, d, axis=ix.ndim - 1)
    pix = jnp.where(lower, fi, bi)
    want_max = (((col // d) ^ (col // k)) & 1) == 0
    hi = jnp.where(want_max, pv, v)
    lo = jnp.where(want_max, v, pv)
    take_p = hi > lo
    return jnp.where(take_p, pv, v), jnp.where(take_p, pix, ix)


def _bitonic_sort_desc(v, ix):
    n = v.shape[-1]
    k = 2
    while k <= n:
        d = k // 2
        while d >= 1:
            v, ix = _bitonic_step(v, ix, d, k)
            d //= 2
        k *= 2
    return v, ix


def _stage_c(s_ref, tv_ref, ti_ref):
    blk = pl.program_id(0)

    def sort_case(W):
        def body():
            v = s_ref[:, :W]
            ix = jax.lax.broadcasted_iota(jnp.int32, (BS, W), 1)
            v, ix = _bitonic_sort_desc(v, ix)
            k = min(W, TOPK)
            tv = v[:, :k]
            tv = jnp.where(tv < -0.5, jnp.float32(-1e9),
                           jnp.where(tv < 0.0, jnp.float32(0.0), tv))
            if W >= TOPK:
                tv_ref[...] = tv
                ti_ref[...] = ix[:, :TOPK]
            else:
                tail = jnp.full((BS, TOPK - W), jnp.float32(-1e9))
                tail_i = (jax.lax.broadcasted_iota(jnp.int32, (BS, TOPK - W), 1)
                          + W)
                tv_ref[...] = jnp.concatenate([tv, tail], axis=1)
                ti_ref[...] = jnp.concatenate([ix, tail_i], axis=1)
        return body

    pl.when(blk == 0)(sort_case(256))
    pl.when(blk == 1)(sort_case(512))
    pl.when(jnp.logical_or(blk == 2, blk == 3))(sort_case(1024))
    pl.when(blk >= 4)(sort_case(2048))


def _rope_tables(cos, sin):
    half = RD // 2
    c = cos[:, :half]
    s = sin[:, :half]
    cosF = jnp.repeat(c, 2, axis=1)
    sinF = jnp.stack([-s, s], axis=-1).reshape(S, RD)
    ones = jnp.ones((S, D - RD), jnp.float32)
    zeros = jnp.zeros((S, D - RD), jnp.float32)
    cos64 = jnp.concatenate([ones, cosF], axis=1)
    sin64 = jnp.concatenate([zeros, sinF], axis=1)
    cosQ = jnp.tile(cos64, (1, H))
    sinQ = jnp.tile(sin64, (1, H))
    return cosQ, sinQ, cos64, sin64


@jax.jit
def kernel(hidden_states, cos, sin, wq, wk, ww):
    hid = hidden_states[0]
    cosQ, sinQ, cosK, sinK = _rope_tables(cos[0], sin[0])

    nblk = S // BQ
    q_rope, k_rope, w = pl.pallas_call(
        _stage_a,
        grid=(nblk,),
        in_specs=[
            pl.BlockSpec((BQ, DM), lambda i: (i, 0)),
            pl.BlockSpec((DM, H * D), lambda i: (0, 0)),
            pl.BlockSpec((DM, D), lambda i: (0, 0)),
            pl.BlockSpec((DM, H), lambda i: (0, 0)),
            pl.BlockSpec((BQ, H * D), lambda i: (i, 0)),
            pl.BlockSpec((BQ, H * D), lambda i: (i, 0)),
            pl.BlockSpec((BQ, D), lambda i: (i, 0)),
            pl.BlockSpec((BQ, D), lambda i: (i, 0)),
        ],
        out_specs=[
            pl.BlockSpec((BQ, H * D), lambda i: (i, 0)),
            pl.BlockSpec((BQ, D), lambda i: (i, 0)),
            pl.BlockSpec((BQ, H), lambda i: (i, 0)),
        ],
        out_shape=[
            jax.ShapeDtypeStruct((S, H * D), jnp.float32),
            jax.ShapeDtypeStruct((S, D), jnp.float32),
            jax.ShapeDtypeStruct((S, H), jnp.float32),
        ],
    )(hid, wq, wk, ww, cosQ, sinQ, cosK, sinK)

    kt = k_rope.T

    scores = pl.pallas_call(
        _stage_b,
        grid=(nblk,),
        in_specs=[
            pl.BlockSpec((BQ, H * D), lambda i: (i, 0)),
            pl.BlockSpec((D, S), lambda i: (0, 0)),
            pl.BlockSpec((BQ, H), lambda i: (i, 0)),
        ],
        out_specs=pl.BlockSpec((BQ, S), lambda i: (i, 0)),
        out_shape=jax.ShapeDtypeStruct((S, S), jnp.float32),
    )(q_rope, kt, w)

    tv, ti = pl.pallas_call(
        _stage_c,
        grid=(S // BS,),
        in_specs=[pl.BlockSpec((BS, S), lambda i: (i, 0))],
        out_specs=[
            pl.BlockSpec((BS, TOPK), lambda i: (i, 0)),
            pl.BlockSpec((BS, TOPK), lambda i: (i, 0)),
        ],
        out_shape=[
            jax.ShapeDtypeStruct((S, TOPK), jnp.float32),
            jax.ShapeDtypeStruct((S, TOPK), jnp.int32),
        ],
    )(scores)
    return tv[None], ti[None]
